# baseline (speedup 1.0000x reference)
import functools

import jax
import jax.numpy as jnp
from jax import lax
from jax.experimental import pallas as pl
from jax.experimental.pallas import tpu as pltpu
from jax.experimental.pallas import tpu_sc as plsc

VOCAB = 100000
DIM = 128
B = 4096
L = 50

NC = 2
NS = 16
NW = NC * NS
SPW = B // NW
CH = 4
NCHUNK = SPW // CH
ROWS = CH * L
GPS = 4
KCH = DIM // 16

_mesh = plsc.VectorSubcoreMesh(core_axis_name="c", subcore_axis_name="s")


@functools.partial(
    pl.kernel,
    mesh=_mesh,
    compiler_params=pltpu.CompilerParams(needs_layout_passes=False),
    out_type=jax.ShapeDtypeStruct((B, DIM), jnp.float32),
    scratch_types=[
        pltpu.VMEM((SPW, L), jnp.int32),
        pltpu.VMEM((SPW, 1), jnp.int32),
        pltpu.VMEM((ROWS + 16, DIM), jnp.float32),
        pltpu.VMEM((ROWS + 16, DIM), jnp.float32),
        pltpu.VMEM((SPW, DIM), jnp.float32),
        pltpu.VMEM((CH, DIM), jnp.float32),
        pltpu.VMEM((CH, DIM), jnp.float32),
        pltpu.SemaphoreType.DMA,
        pltpu.SemaphoreType.DMA,
        pltpu.SemaphoreType.DMA,
        pltpu.SemaphoreType.DMA,
    ],
)
def _w2v(target_hbm, context_hbm, w_in_hbm, w_out_hbm, out_hbm,
         cidx_v, tidx_v, crow0_v, crow1_v, trow_v,
         outc0_v, outc1_v, sem0, sem1, osem0, osem1):
    wid = lax.axis_index("s") * NC + lax.axis_index("c")
    base = wid * SPW
    lanes = lax.iota(jnp.int32, 16)
    pltpu.sync_copy(
        context_hbm.at[pl.ds(pl.multiple_of(base, 8), SPW)], cidx_v)
    pltpu.sync_copy(target_hbm.at[pl.ds(pl.multiple_of(base, 8), SPW)],
                    tidx_v)
    tdescs = [
        pltpu.make_async_copy(
            w_in_hbm.at[tidx_v.at[r]], trow_v.at[pl.ds(r, 1)], sem0)
        for r in range(SPW)
    ]
    for c in tdescs:
        c.start()
    for c in tdescs:
        c.wait()

    bufs = ((crow0_v, outc0_v, sem0, osem0),
            (crow1_v, outc1_v, sem1, osem1))

    def _descs(ci, b):
        crow_b, _, sem_b, _ = bufs[b]
        return [
            pltpu.make_async_copy(
                w_out_hbm.at[cidx_v.at[ci * CH + s]],
                crow_b.at[pl.ds(s * L, L)],
                sem_b,
            )
            for s in range(CH)
        ]

    def _issue(ci, b):
        for c in _descs(ci, b):
            c.start()

    def _drain(ci, b):
        for c in _descs(ci, b):
            c.wait()

    def _out_descs(ci, b):
        _, outc_b, _, osem_b = bufs[b]
        samp0 = base + ci * CH
        return [
            pltpu.make_async_copy(
                outc_b,
                out_hbm.at[pl.ds(pl.multiple_of(samp0, 4), CH)],
                osem_b)
        ]

    def _compute(ci, b):
        crow_b, outc_b, _, _ = bufs[b]

        def samp_body(s, carry2):
            row0 = s * L
            si = ci * CH + s
            tvecs = [trow_v[si, pl.ds(k * 16, 16)] for k in range(KCH)]
            zero = jnp.zeros((16,), jnp.float32)

            def do_group(g, nrows):
                base_r = row0 + g * 16
                res = zero
                for rr in range(nrows):
                    r = base_r + rr
                    acc = tvecs[0] * crow_b[r, pl.ds(0, 16)]
                    for k in range(1, KCH):
                        acc = acc + tvecs[k] * crow_b[r, pl.ds(k * 16, 16)]
                    sval = jnp.sum(acc)
                    res = jnp.where(lanes == rr, sval, res)
                outc_b[s, pl.ds(g * 16, 16)] = res

            for g in range(3):
                do_group(g, 16)
            do_group(3, 2)
            return carry2

        lax.fori_loop(0, CH, samp_body, 0, unroll=1)
        for c in _out_descs(ci, b):
            c.start()

    def _out_drain(ci, b):
        for c in _out_descs(ci, b):
            c.wait()

    _issue(0, 0)

    def pair_body(i, carry):
        ci = 2 * i
        _issue(ci + 1, 1)
        _drain(ci, 0)

        @pl.when(i > 0)
        def _():
            _out_drain(ci - 2, 0)

        _compute(ci, 0)

        @pl.when(i + 1 < NCHUNK // 2)
        def _():
            _issue(ci + 2, 0)

        _drain(ci + 1, 1)

        @pl.when(i > 0)
        def _():
            _out_drain(ci - 1, 1)

        _compute(ci + 1, 1)
        return carry

    lax.fori_loop(0, NCHUNK // 2, pair_body, 0, unroll=1)
    _out_drain(NCHUNK - 2, 0)
    _out_drain(NCHUNK - 1, 1)


def kernel(target, context, W_in, W_out):
    out = _w2v(target, context, W_in, W_out)
    return out[:, :L]


# speedup vs baseline: 1.0268x; 1.0268x over previous
import functools

import jax
import jax.numpy as jnp
from jax import lax
from jax.experimental import pallas as pl
from jax.experimental.pallas import tpu as pltpu
from jax.experimental.pallas import tpu_sc as plsc

VOCAB = 100000
DIM = 128
B = 4096
L = 50

NC = 2
NS = 16
NW = NC * NS
SPW = B // NW
CH = 4
NCHUNK = SPW // CH
ROWS = CH * L
GPS = 4
KCH = DIM // 16

_mesh = plsc.VectorSubcoreMesh(core_axis_name="c", subcore_axis_name="s")


@functools.partial(
    pl.kernel,
    mesh=_mesh,
    compiler_params=pltpu.CompilerParams(needs_layout_passes=False,
                                         skip_device_barrier=True),
    out_type=jax.ShapeDtypeStruct((B, DIM), jnp.float32),
    scratch_types=[
        pltpu.VMEM((SPW, L), jnp.int32),
        pltpu.VMEM((SPW,), jnp.int32),
        pltpu.VMEM((ROWS + 16, DIM), jnp.float32),
        pltpu.VMEM((ROWS + 16, DIM), jnp.float32),
        pltpu.VMEM((SPW, DIM), jnp.float32),
        pltpu.VMEM((CH, DIM), jnp.float32),
        pltpu.VMEM((CH, DIM), jnp.float32),
        pltpu.SemaphoreType.DMA,
        pltpu.SemaphoreType.DMA,
        pltpu.SemaphoreType.DMA,
        pltpu.SemaphoreType.DMA,
    ],
)
def _w2v(target_hbm, context_hbm, w_in_hbm, w_out_hbm, out_hbm,
         cidx_v, tidx_v, crow0_v, crow1_v, trow_v,
         outc0_v, outc1_v, sem0, sem1, osem0, osem1):
    wid = lax.axis_index("s") * NC + lax.axis_index("c")
    base = wid * SPW
    lanes = lax.iota(jnp.int32, 16)
    pltpu.sync_copy(
        context_hbm.at[pl.ds(pl.multiple_of(base, 8), SPW)], cidx_v)
    pltpu.sync_copy(target_hbm.at[pl.ds(pl.multiple_of(base, 8), SPW)],
                    tidx_v)
    pltpu.async_copy(w_in_hbm.at[tidx_v], trow_v, sem0).wait()

    bufs = ((crow0_v, outc0_v, sem0, osem0),
            (crow1_v, outc1_v, sem1, osem1))

    def _descs(ci, b):
        crow_b, _, sem_b, _ = bufs[b]
        return [
            pltpu.make_async_copy(
                w_out_hbm.at[cidx_v.at[ci * CH + s]],
                crow_b.at[pl.ds(s * L, L)],
                sem_b,
            )
            for s in range(CH)
        ]

    def _issue(ci, b):
        for c in _descs(ci, b):
            c.start()

    def _drain(ci, b):
        for c in _descs(ci, b):
            c.wait()

    def _out_descs(ci, b):
        _, outc_b, _, osem_b = bufs[b]
        samp0 = base + ci * CH
        return [
            pltpu.make_async_copy(
                outc_b,
                out_hbm.at[pl.ds(pl.multiple_of(samp0, 4), CH)],
                osem_b)
        ]

    def _compute(ci, b):
        crow_b, outc_b, _, _ = bufs[b]

        def samp_body(s, carry2):
            row0 = s * L
            si = ci * CH + s
            tvecs = [trow_v[si, pl.ds(k * 16, 16)] for k in range(KCH)]
            zero = jnp.zeros((16,), jnp.float32)

            def do_group(g, nrows):
                base_r = row0 + g * 16
                res = zero
                for rr in range(nrows):
                    r = base_r + rr
                    acc = tvecs[0] * crow_b[r, pl.ds(0, 16)]
                    for k in range(1, KCH):
                        acc = acc + tvecs[k] * crow_b[r, pl.ds(k * 16, 16)]
                    sval = jnp.sum(acc)
                    res = jnp.where(lanes == rr, sval, res)
                outc_b[s, pl.ds(g * 16, 16)] = res

            for g in range(3):
                do_group(g, 16)
            do_group(3, 2)
            return carry2

        lax.fori_loop(0, CH, samp_body, 0, unroll=1)
        for c in _out_descs(ci, b):
            c.start()

    def _out_drain(ci, b):
        for c in _out_descs(ci, b):
            c.wait()

    _issue(0, 0)

    def pair_body(i, carry):
        ci = 2 * i
        _issue(ci + 1, 1)
        _drain(ci, 0)

        @pl.when(i > 0)
        def _():
            _out_drain(ci - 2, 0)

        _compute(ci, 0)

        @pl.when(i + 1 < NCHUNK // 2)
        def _():
            _issue(ci + 2, 0)

        _drain(ci + 1, 1)

        @pl.when(i > 0)
        def _():
            _out_drain(ci - 1, 1)

        _compute(ci + 1, 1)
        return carry

    lax.fori_loop(0, NCHUNK // 2, pair_body, 0, unroll=1)
    _out_drain(NCHUNK - 2, 0)
    _out_drain(NCHUNK - 1, 1)


def kernel(target, context, W_in, W_out):
    out = _w2v(target.reshape(B), context, W_in, W_out)
    return out[:, :L]


# speedup vs baseline: 1.0308x; 1.0038x over previous
import functools

import jax
import jax.numpy as jnp
from jax import lax
from jax.experimental import pallas as pl
from jax.experimental.pallas import tpu as pltpu
from jax.experimental.pallas import tpu_sc as plsc

VOCAB = 100000
DIM = 128
B = 4096
L = 50

NC = 2
NS = 16
NW = NC * NS
SPW = B // NW
CH = 4
NCHUNK = SPW // CH
ROWS = CH * L
GPS = 4
KCH = DIM // 16

_mesh = plsc.VectorSubcoreMesh(core_axis_name="c", subcore_axis_name="s")


@functools.partial(
    pl.kernel,
    mesh=_mesh,
    compiler_params=pltpu.CompilerParams(needs_layout_passes=False),
    out_type=jax.ShapeDtypeStruct((B, DIM), jnp.float32),
    scratch_types=[
        pltpu.VMEM((SPW, L), jnp.int32),
        pltpu.VMEM((SPW,), jnp.int32),
        pltpu.VMEM((ROWS + 16, DIM), jnp.float32),
        pltpu.VMEM((ROWS + 16, DIM), jnp.float32),
        pltpu.VMEM((SPW, DIM), jnp.float32),
        pltpu.VMEM((CH, DIM), jnp.float32),
        pltpu.VMEM((CH, DIM), jnp.float32),
        pltpu.SemaphoreType.DMA,
        pltpu.SemaphoreType.DMA,
        pltpu.SemaphoreType.DMA,
        pltpu.SemaphoreType.DMA,
    ],
)
def _w2v(target_hbm, context_hbm, w_in_hbm, w_out_hbm, out_hbm,
         cidx_v, tidx_v, crow0_v, crow1_v, trow_v,
         outc0_v, outc1_v, sem0, sem1, osem0, osem1):
    wid = lax.axis_index("s") * NC + lax.axis_index("c")
    base = wid * SPW
    lanes = lax.iota(jnp.int32, 16)
    pltpu.sync_copy(
        context_hbm.at[pl.ds(pl.multiple_of(base, 8), SPW)], cidx_v)
    pltpu.sync_copy(target_hbm.at[pl.ds(pl.multiple_of(base, 8), SPW)],
                    tidx_v)
    pltpu.async_copy(w_in_hbm.at[tidx_v], trow_v, sem0).wait()

    bufs = ((crow0_v, outc0_v, sem0, osem0),
            (crow1_v, outc1_v, sem1, osem1))

    def _descs(ci, b):
        crow_b, _, sem_b, _ = bufs[b]
        return [
            pltpu.make_async_copy(
                w_out_hbm.at[cidx_v.at[ci * CH + s]],
                crow_b.at[pl.ds(s * L, L)],
                sem_b,
            )
            for s in range(CH)
        ]

    def _issue(ci, b):
        for c in _descs(ci, b):
            c.start()

    def _drain(ci, b):
        for c in _descs(ci, b):
            c.wait()

    def _out_descs(ci, b):
        _, outc_b, _, osem_b = bufs[b]
        samp0 = base + ci * CH
        return [
            pltpu.make_async_copy(
                outc_b,
                out_hbm.at[pl.ds(pl.multiple_of(samp0, 4), CH)],
                osem_b)
        ]

    def _compute(ci, b):
        crow_b, outc_b, _, _ = bufs[b]

        def samp_body(s, carry2):
            row0 = s * L
            si = ci * CH + s
            tvecs = [trow_v[si, pl.ds(k * 16, 16)] for k in range(KCH)]
            zero = jnp.zeros((16,), jnp.float32)

            def do_group(g, nrows):
                base_r = row0 + g * 16
                res = zero
                for rr in range(nrows):
                    r = base_r + rr
                    acc = tvecs[0] * crow_b[r, pl.ds(0, 16)]
                    for k in range(1, KCH):
                        acc = acc + tvecs[k] * crow_b[r, pl.ds(k * 16, 16)]
                    sval = jnp.sum(acc)
                    res = jnp.where(lanes == rr, sval, res)
                outc_b[s, pl.ds(g * 16, 16)] = res

            for g in range(3):
                do_group(g, 16)
            do_group(3, 2)
            return carry2

        lax.fori_loop(0, CH, samp_body, 0, unroll=1)
        for c in _out_descs(ci, b):
            c.start()

    def _out_drain(ci, b):
        for c in _out_descs(ci, b):
            c.wait()

    _issue(0, 0)

    def pair_body(i, carry):
        ci = 2 * i
        _issue(ci + 1, 1)
        _drain(ci, 0)

        @pl.when(i > 0)
        def _():
            _out_drain(ci - 2, 0)

        _compute(ci, 0)

        @pl.when(i + 1 < NCHUNK // 2)
        def _():
            _issue(ci + 2, 0)

        _drain(ci + 1, 1)

        @pl.when(i > 0)
        def _():
            _out_drain(ci - 1, 1)

        _compute(ci + 1, 1)
        return carry

    lax.fori_loop(0, NCHUNK // 2, pair_body, 0, unroll=1)
    _out_drain(NCHUNK - 2, 0)
    _out_drain(NCHUNK - 1, 1)


def kernel(target, context, W_in, W_out):
    out = _w2v(target.reshape(B), context, W_in, W_out)
    return out[:, :L]
